# Initial kernel scaffold; baseline (speedup 1.0000x reference)
import jax
import jax.numpy as jnp
from jax.experimental import pallas as pl


def kernel(x, position_ids, cos_cached, sin_cached):
    raise NotImplementedError("write your pallas kernel here")


# speedup vs baseline: 3.5652x; 3.5652x over previous
import functools

import jax
import jax.numpy as jnp
from jax import lax
from jax.experimental import pallas as pl
from jax.experimental.pallas import tpu as pltpu
from jax.experimental.pallas import tpu_sc as plsc

DIM = 128
NC = 2
NS = 16
NW = NC * NS
CHUNK = 128


@functools.partial(jax.jit, static_argnums=(3, 4))
def _gather_pairs(pos3, cos_cached, sin_cached, n_idx, n_chunks):
    mesh = plsc.VectorSubcoreMesh(
        core_axis_name="c", subcore_axis_name="s",
        num_cores=NC, num_subcores=NS)

    out_type = (
        jax.ShapeDtypeStruct((n_idx, DIM), jnp.float32),
        jax.ShapeDtypeStruct((n_idx, DIM), jnp.float32),
    )

    @functools.partial(
        pl.kernel,
        out_type=out_type,
        mesh=mesh,
        scratch_types=[
            pltpu.VMEM((n_chunks, CHUNK), jnp.int32),
            pltpu.VMEM((CHUNK, DIM), jnp.float32),
            pltpu.VMEM((CHUNK, DIM), jnp.float32),
            pltpu.SemaphoreType.DMA,
            pltpu.SemaphoreType.DMA,
        ],
    )
    def k(pos_hbm, cos_hbm, sin_hbm, cos_out, sin_out,
          idx_v, rows_c, rows_s, sem_c, sem_s):
        wid = lax.axis_index("s") * NC + lax.axis_index("c")
        pltpu.sync_copy(pos_hbm.at[wid], idx_v)
        base = wid * (n_chunks * CHUNK)
        for ch in range(n_chunks):
            cpy_c = pltpu.async_copy(cos_hbm.at[idx_v.at[ch]], rows_c, sem_c)
            cpy_s = pltpu.async_copy(sin_hbm.at[idx_v.at[ch]], rows_s, sem_s)
            cpy_c.wait()
            pltpu.sync_copy(rows_c, cos_out.at[pl.ds(base + ch * CHUNK, CHUNK)])
            cpy_s.wait()
            pltpu.sync_copy(rows_s, sin_out.at[pl.ds(base + ch * CHUNK, CHUNK)])

    return k(pos3, cos_cached, sin_cached)


def kernel(x, position_ids, cos_cached, sin_cached):
    B, S = position_ids.shape
    n_idx = B * S
    n_chunks = n_idx // (NW * CHUNK)
    pos3 = position_ids.astype(jnp.int32).reshape(NW, n_chunks, CHUNK)
    cos_rows, sin_rows = _gather_pairs(
        pos3, cos_cached, sin_cached, n_idx, n_chunks)
    return (cos_rows.reshape(B, 1, S, DIM), sin_rows.reshape(B, 1, S, DIM))


# speedup vs baseline: 3.8230x; 1.0723x over previous
import functools

import jax
import jax.numpy as jnp
from jax import lax
from jax.experimental import pallas as pl
from jax.experimental.pallas import tpu as pltpu
from jax.experimental.pallas import tpu_sc as plsc

DIM = 128
NC = 2
NS = 16
NW = NC * NS
CHUNK = 128


@functools.partial(jax.jit, static_argnums=(3, 4))
def _gather_pairs(pos3, cos_cached, sin_cached, n_idx, n_chunks):
    mesh = plsc.VectorSubcoreMesh(
        core_axis_name="c", subcore_axis_name="s",
        num_cores=NC, num_subcores=NS)

    out_type = (
        jax.ShapeDtypeStruct((n_idx, DIM), jnp.float32),
        jax.ShapeDtypeStruct((n_idx, DIM), jnp.float32),
    )

    NBUF = 3

    @functools.partial(
        pl.kernel,
        out_type=out_type,
        mesh=mesh,
        scratch_types=[
            pltpu.VMEM((n_chunks, CHUNK), jnp.int32),
            [pltpu.VMEM((CHUNK, DIM), jnp.float32) for _ in range(NBUF)],
            [pltpu.VMEM((CHUNK, DIM), jnp.float32) for _ in range(NBUF)],
            [pltpu.SemaphoreType.DMA for _ in range(4 * NBUF)],
        ],
    )
    def k(pos_hbm, cos_hbm, sin_hbm, cos_out, sin_out,
          idx_v, rows_c, rows_s, sems):
        gc, gs = sems[:NBUF], sems[NBUF:2 * NBUF]
        wc, ws = sems[2 * NBUF:3 * NBUF], sems[3 * NBUF:]
        wid = lax.axis_index("s") * NC + lax.axis_index("c")
        pltpu.sync_copy(pos_hbm.at[wid], idx_v)
        base = wid * (n_chunks * CHUNK)

        def gather(ch):
            b = ch % NBUF
            return (
                pltpu.async_copy(cos_hbm.at[idx_v.at[ch]], rows_c[b], gc[b]),
                pltpu.async_copy(sin_hbm.at[idx_v.at[ch]], rows_s[b], gs[b]),
            )

        g = {ch: gather(ch) for ch in range(min(NBUF, n_chunks))}
        w = {}
        for ch in range(n_chunks):
            b = ch % NBUF
            out_slice = pl.ds(base + ch * CHUNK, CHUNK)
            g[ch][0].wait()
            w[ch] = [pltpu.async_copy(rows_c[b], cos_out.at[out_slice], wc[b])]
            g[ch][1].wait()
            w[ch].append(pltpu.async_copy(rows_s[b], sin_out.at[out_slice], ws[b]))
            nxt = ch + NBUF
            if nxt < n_chunks:
                for cpy in w[ch]:
                    cpy.wait()
                del w[ch]
                g[nxt] = gather(nxt)
        for ch in sorted(w):
            for cpy in w[ch]:
                cpy.wait()

    return k(pos3, cos_cached, sin_cached)


def kernel(x, position_ids, cos_cached, sin_cached):
    B, S = position_ids.shape
    n_idx = B * S
    n_chunks = n_idx // (NW * CHUNK)
    pos3 = position_ids.astype(jnp.int32).reshape(NW, n_chunks, CHUNK)
    cos_rows, sin_rows = _gather_pairs(
        pos3, cos_cached, sin_cached, n_idx, n_chunks)
    return (cos_rows.reshape(B, 1, S, DIM), sin_rows.reshape(B, 1, S, DIM))
